# baseline (speedup 1.0000x reference)
import functools

import jax
import jax.numpy as jnp
from jax import lax
from jax.experimental import pallas as pl
from jax.experimental.pallas import tpu as pltpu
from jax.experimental.pallas import tpu_sc as plsc

NC = 2
NS = 16
NW = NC * NS
CHUNK = 128

_MESH = plsc.VectorSubcoreMesh(
    core_axis_name="c", subcore_axis_name="s", num_cores=NC, num_subcores=NS
)


K = 8
CORE0_SHARE = 1.0


def _slab(c, s, n0, n1):
  is0 = c == 0
  nmine = jnp.where(is0, n0, n1)
  chunk0 = jnp.where(is0, s * n0, NS * n0 + s * n1)
  groups = jnp.where(is0, n0 // K, n1 // K)
  return chunk0, nmine, groups


def _splits(e_pad):
  per_pair = e_pad // (CHUNK * NS)
  n0 = max(K, (int(per_pair * CORE0_SHARE) // K) * K)
  n1 = per_pair - n0
  return n0, n1


SLAB = 64


def _make_deg_pass(n_pad, e_pad):
  n0, n1 = _splits(e_pad)
  rpt = n_pad // NS

  @functools.partial(
      pl.kernel,
      out_type=jax.ShapeDtypeStruct((NC, n_pad, 16), jnp.float32),
      mesh=_MESH,
      scratch_types=[
          pltpu.VMEM((SLAB, CHUNK), jnp.int32),
          pltpu.VMEM((CHUNK, 16), jnp.float32),
          pltpu.VMEM_SHARED((n_pad, 16), jnp.float32),
          pltpu.SemaphoreType.DMA,
      ],
      compiler_params=pltpu.CompilerParams(use_tc_tiling_on_sc=False),
  )
  def deg_pass(dst_hbm, ones_hbm, zeros_hbm, out_hbm, didx_a, ones_v, acc_sh, sem):
    c = lax.axis_index("c")
    s = lax.axis_index("s")
    chunk0, nmine, _ = _slab(c, s, n0, n1)
    r0 = s * rpt
    pltpu.sync_copy(zeros_hbm, acc_sh.at[pl.ds(r0, rpt)])
    pltpu.sync_copy(ones_hbm, ones_v)
    plsc.subcore_barrier()

    @pl.loop(0, (nmine + SLAB - 1) // SLAB)
    def _half(h):
      sl0 = chunk0 + h * SLAB
      pltpu.sync_copy(dst_hbm.at[pl.ds(sl0, SLAB)], didx_a)

      @pl.loop(0, jnp.minimum(SLAB, nmine - h * SLAB) // K)
      def _grp(g):
        base = g * K
        cps = [
            pltpu.async_copy(ones_v, acc_sh.at[didx_a.at[base + b]], sem,
                             add=True)
            for b in range(K)
        ]
        for cp in cps:
          cp.wait()

    plsc.subcore_barrier()
    pltpu.sync_copy(acc_sh.at[pl.ds(r0, rpt)], out_hbm.at[c, pl.ds(r0, rpt)])

  return deg_pass


def _make_edge_pass(n_pad, e_pad, width):
  n0, n1 = _splits(e_pad)
  rpt = n_pad // NS

  @functools.partial(
      pl.kernel,
      out_type=jax.ShapeDtypeStruct((NC, n_pad, width), jnp.float32),
      mesh=_MESH,
      scratch_types=[
          pltpu.VMEM((SLAB, CHUNK), jnp.int32),
          pltpu.VMEM((SLAB, CHUNK), jnp.int32),
          pltpu.VMEM((K, CHUNK, width), jnp.float32),
          pltpu.VMEM_SHARED((n_pad, width), jnp.float32),
          pltpu.SemaphoreType.DMA,
          pltpu.SemaphoreType.DMA,
      ],
      compiler_params=pltpu.CompilerParams(use_tc_tiling_on_sc=False),
  )
  def edge_pass(t_hbm, src_hbm, dst_hbm, zeros_hbm, out_hbm,
                sidx_a, didx_a, rows_v, acc_sh, sem_g, sem_s):
    c = lax.axis_index("c")
    s = lax.axis_index("s")
    chunk0, nmine, _ = _slab(c, s, n0, n1)
    r0 = s * rpt
    pltpu.sync_copy(zeros_hbm, acc_sh.at[pl.ds(r0, rpt)])
    plsc.subcore_barrier()

    @pl.loop(0, (nmine + SLAB - 1) // SLAB)
    def _half(h):
      sl0 = chunk0 + h * SLAB
      pltpu.sync_copy(src_hbm.at[pl.ds(sl0, SLAB)], sidx_a)
      pltpu.sync_copy(dst_hbm.at[pl.ds(sl0, SLAB)], didx_a)

      @pl.loop(0, jnp.minimum(SLAB, nmine - h * SLAB) // K)
      def _grp(g):
        base = g * K
        gcs = [
            pltpu.async_copy(t_hbm.at[sidx_a.at[base + b]], rows_v.at[b], sem_g)
            for b in range(K)
        ]
        for cp in gcs:
          cp.wait()
        scs = [
            pltpu.async_copy(rows_v.at[b], acc_sh.at[didx_a.at[base + b]],
                             sem_s, add=True)
            for b in range(K)
        ]
        for cp in scs:
          cp.wait()

    plsc.subcore_barrier()
    pltpu.sync_copy(acc_sh.at[pl.ds(r0, rpt)], out_hbm.at[c, pl.ds(r0, rpt)])

  return edge_pass


def _tc1_body(x_ref, w1_ref, b1_ref, wu_ref, x1_ref, u_ref):
  xb = x_ref[...]
  x1_ref[...] = jnp.maximum(xb @ w1_ref[...] + b1_ref[...], 0.0)
  u_ref[...] = xb @ wu_ref[...]


def _tc2_body(degp_ref, u_ref, t1_ref, dinv_ref):
  deg = degp_ref[0] + degp_ref[1] + 1.0
  dinv = 1.0 / jnp.sqrt(deg)
  dinv_ref[...] = dinv
  t1_ref[...] = u_ref[...] * dinv[:, 0:1]


def _tc3_body(z1_ref, t1_ref, dinv_ref, b2_ref, x2_ref, t2_ref):
  dinv = dinv_ref[:, 0:1]
  h = t1_ref.shape[1] // 2
  zsum = z1_ref[0] + z1_ref[1] + t1_ref[...]
  x2_ref[...] = jnp.maximum(zsum[:, :h] * dinv + b2_ref[...], 0.0)
  t2_ref[...] = zsum[:, h:] * (dinv * dinv)


def _tc4_body(z2_ref, t2_ref, dinv_ref, x1_ref, x2_ref, b3_ref,
              wl1_ref, wl2_ref, wl3_ref, bl_ref, out_ref):
  dinv = dinv_ref[:, 0:1]
  y3 = (z2_ref[0] + z2_ref[1] + t2_ref[...]) * dinv
  x3 = jnp.maximum(y3 + b3_ref[...], 0.0)
  z = (x1_ref[...] @ wl1_ref[...] + x2_ref[...] @ wl2_ref[...]
       + x3 @ wl3_ref[...] + bl_ref[...])
  m = jnp.max(z, axis=1, keepdims=True)
  out_ref[...] = z - (m + jnp.log(jnp.sum(jnp.exp(z - m), axis=1, keepdims=True)))


def _full(shape):
  return pl.BlockSpec(shape, lambda i: (0,) * len(shape))


def kernel(x, edge_index, W1, b1, W2, b2, W3, b3, Wl, bl):
  n, d_feat = x.shape
  hidden = W1.shape[1]
  n_classes = Wl.shape[1]
  e = edge_index.shape[1]

  n_pad = ((n + 2047) // 2048) * 2048
  egrp = NS * CHUNK * K
  e_pad = ((e + egrp - 1) // egrp) * egrp
  rpt = n_pad // NS
  total_chunks = e_pad // CHUNK

  src = jnp.concatenate([edge_index[0], jnp.full((e_pad - e + egrp,), n, jnp.int32)])
  dst = jnp.concatenate([edge_index[1], jnp.full((e_pad - e + egrp,), n, jnp.int32)])
  src = src.reshape(total_chunks + NS * K, CHUNK)
  dst = dst.reshape(total_chunks + NS * K, CHUNK)

  xp = jnp.concatenate([x, jnp.zeros((n_pad - n, d_feat), x.dtype)])
  wu = jnp.concatenate([W2, W3], axis=1)
  ones16 = jnp.ones((CHUNK, 16), jnp.float32)
  z16 = jnp.zeros((rpt, 16), jnp.float32)
  z32 = jnp.zeros((rpt, 32), jnp.float32)
  z64 = jnp.zeros((rpt, 64), jnp.float32)

  rblk = 1024
  ngrid = n_pad // rblk

  x1p, up = pl.pallas_call(
      _tc1_body,
      grid=(ngrid,),
      in_specs=[
          pl.BlockSpec((rblk, d_feat), lambda i: (i, 0)),
          _full((d_feat, hidden)),
          _full((1, hidden)),
          _full((d_feat, 2 * hidden)),
      ],
      out_specs=[
          pl.BlockSpec((rblk, hidden), lambda i: (i, 0)),
          pl.BlockSpec((rblk, 2 * hidden), lambda i: (i, 0)),
      ],
      out_shape=[
          jax.ShapeDtypeStruct((n_pad, hidden), jnp.float32),
          jax.ShapeDtypeStruct((n_pad, 2 * hidden), jnp.float32),
      ],
  )(xp, W1, b1.reshape(1, hidden), wu)

  degp = _make_deg_pass(n_pad, e_pad)(dst, ones16, z16)

  t1p, dinvb = pl.pallas_call(
      _tc2_body,
      grid=(ngrid,),
      in_specs=[
          pl.BlockSpec((2, rblk, 16), lambda i: (0, i, 0)),
          pl.BlockSpec((rblk, 2 * hidden), lambda i: (i, 0)),
      ],
      out_specs=[
          pl.BlockSpec((rblk, 2 * hidden), lambda i: (i, 0)),
          pl.BlockSpec((rblk, 16), lambda i: (i, 0)),
      ],
      out_shape=[
          jax.ShapeDtypeStruct((n_pad, 2 * hidden), jnp.float32),
          jax.ShapeDtypeStruct((n_pad, 16), jnp.float32),
      ],
  )(degp, up)

  z1p = _make_edge_pass(n_pad, e_pad, 2 * hidden)(t1p, src, dst, z64)

  x2p, t2p = pl.pallas_call(
      _tc3_body,
      grid=(ngrid,),
      in_specs=[
          pl.BlockSpec((2, rblk, 2 * hidden), lambda i: (0, i, 0)),
          pl.BlockSpec((rblk, 2 * hidden), lambda i: (i, 0)),
          pl.BlockSpec((rblk, 16), lambda i: (i, 0)),
          _full((1, hidden)),
      ],
      out_specs=[
          pl.BlockSpec((rblk, hidden), lambda i: (i, 0)),
          pl.BlockSpec((rblk, hidden), lambda i: (i, 0)),
      ],
      out_shape=[
          jax.ShapeDtypeStruct((n_pad, hidden), jnp.float32),
          jax.ShapeDtypeStruct((n_pad, hidden), jnp.float32),
      ],
  )(z1p, t1p, dinvb, b2.reshape(1, hidden))

  z2p = _make_edge_pass(n_pad, e_pad, hidden)(t2p, src, dst, z32)

  rblk4 = 1000
  ngrid4 = n // rblk4
  out = pl.pallas_call(
      _tc4_body,
      grid=(ngrid4,),
      in_specs=[
          pl.BlockSpec((2, rblk4, hidden), lambda i: (0, i, 0)),
          pl.BlockSpec((rblk4, hidden), lambda i: (i, 0)),
          pl.BlockSpec((rblk4, 16), lambda i: (i, 0)),
          pl.BlockSpec((rblk4, hidden), lambda i: (i, 0)),
          pl.BlockSpec((rblk4, hidden), lambda i: (i, 0)),
          _full((1, hidden)),
          _full((hidden, n_classes)),
          _full((hidden, n_classes)),
          _full((hidden, n_classes)),
          _full((1, n_classes)),
      ],
      out_specs=pl.BlockSpec((rblk4, n_classes), lambda i: (i, 0)),
      out_shape=jax.ShapeDtypeStruct((n, n_classes), jnp.float32),
  )(z2p, t2p, dinvb, x1p, x2p, b3.reshape(1, hidden),
    Wl[:hidden], Wl[hidden:2 * hidden], Wl[2 * hidden:], bl.reshape(1, n_classes))

  return out


# speedup vs baseline: 1.3147x; 1.3147x over previous
import functools

import jax
import jax.numpy as jnp
from jax import lax
from jax.experimental import pallas as pl
from jax.experimental.pallas import tpu as pltpu
from jax.experimental.pallas import tpu_sc as plsc

NC = 2
NS = 16
NW = NC * NS
CHUNK = 128

_MESH = plsc.VectorSubcoreMesh(
    core_axis_name="c", subcore_axis_name="s", num_cores=NC, num_subcores=NS
)


K = 8
CORE0_SHARE = 0.85


def _slab(c, s, n0, n1):
  is0 = c == 0
  nmine = jnp.where(is0, n0, n1)
  chunk0 = jnp.where(is0, s * n0, NS * n0 + s * n1)
  groups = jnp.where(is0, n0 // K, n1 // K)
  return chunk0, nmine, groups


def _splits(e_pad):
  per_pair = e_pad // (CHUNK * NS)
  n0 = max(K, (int(per_pair * CORE0_SHARE) // K) * K)
  n1 = per_pair - n0
  return n0, n1


SLAB = 64


def _make_deg_pass(n_pad, e_pad):
  n0, n1 = _splits(e_pad)
  rpt = n_pad // NS

  @functools.partial(
      pl.kernel,
      out_type=jax.ShapeDtypeStruct((NC, n_pad, 16), jnp.float32),
      mesh=_MESH,
      scratch_types=[
          pltpu.VMEM((SLAB, CHUNK), jnp.int32),
          pltpu.VMEM((CHUNK, 16), jnp.float32),
          pltpu.VMEM_SHARED((n_pad, 16), jnp.float32),
          pltpu.SemaphoreType.DMA,
      ],
      compiler_params=pltpu.CompilerParams(use_tc_tiling_on_sc=False),
  )
  def deg_pass(dst_hbm, ones_hbm, zeros_hbm, out_hbm, didx_a, ones_v, acc_sh, sem):
    c = lax.axis_index("c")
    s = lax.axis_index("s")
    chunk0, nmine, _ = _slab(c, s, n0, n1)
    r0 = s * rpt
    pltpu.sync_copy(zeros_hbm, acc_sh.at[pl.ds(r0, rpt)])
    pltpu.sync_copy(ones_hbm, ones_v)
    plsc.subcore_barrier()

    @pl.loop(0, (nmine + SLAB - 1) // SLAB)
    def _half(h):
      sl0 = chunk0 + h * SLAB
      pltpu.sync_copy(dst_hbm.at[pl.ds(sl0, SLAB)], didx_a)

      @pl.loop(0, jnp.minimum(SLAB, nmine - h * SLAB) // K)
      def _grp(g):
        base = g * K
        cps = [
            pltpu.async_copy(ones_v, acc_sh.at[didx_a.at[base + b]], sem,
                             add=True)
            for b in range(K)
        ]
        for cp in cps:
          cp.wait()

    plsc.subcore_barrier()
    pltpu.sync_copy(acc_sh.at[pl.ds(r0, rpt)], out_hbm.at[c, pl.ds(r0, rpt)])

  return deg_pass


def _make_edge_pass(n_pad, e_pad, width):
  n0, n1 = _splits(e_pad)
  rpt = n_pad // NS

  @functools.partial(
      pl.kernel,
      out_type=jax.ShapeDtypeStruct((NC, n_pad, width), jnp.float32),
      mesh=_MESH,
      scratch_types=[
          pltpu.VMEM((SLAB, CHUNK), jnp.int32),
          pltpu.VMEM((SLAB, CHUNK), jnp.int32),
          pltpu.VMEM((K, CHUNK, width), jnp.float32),
          pltpu.VMEM_SHARED((n_pad, width), jnp.float32),
          pltpu.SemaphoreType.DMA,
          pltpu.SemaphoreType.DMA,
      ],
      compiler_params=pltpu.CompilerParams(use_tc_tiling_on_sc=False),
  )
  def edge_pass(t_hbm, src_hbm, dst_hbm, zeros_hbm, out_hbm,
                sidx_a, didx_a, rows_v, acc_sh, sem_g, sem_s):
    c = lax.axis_index("c")
    s = lax.axis_index("s")
    chunk0, nmine, _ = _slab(c, s, n0, n1)
    r0 = s * rpt
    pltpu.sync_copy(zeros_hbm, acc_sh.at[pl.ds(r0, rpt)])
    plsc.subcore_barrier()

    @pl.loop(0, (nmine + SLAB - 1) // SLAB)
    def _half(h):
      sl0 = chunk0 + h * SLAB
      pltpu.sync_copy(src_hbm.at[pl.ds(sl0, SLAB)], sidx_a)
      pltpu.sync_copy(dst_hbm.at[pl.ds(sl0, SLAB)], didx_a)

      @pl.loop(0, jnp.minimum(SLAB, nmine - h * SLAB) // K)
      def _grp(g):
        base = g * K
        gcs = [
            pltpu.async_copy(t_hbm.at[sidx_a.at[base + b]], rows_v.at[b], sem_g)
            for b in range(K)
        ]
        for cp in gcs:
          cp.wait()
        scs = [
            pltpu.async_copy(rows_v.at[b], acc_sh.at[didx_a.at[base + b]],
                             sem_s, add=True)
            for b in range(K)
        ]
        for cp in scs:
          cp.wait()

    plsc.subcore_barrier()
    pltpu.sync_copy(acc_sh.at[pl.ds(r0, rpt)], out_hbm.at[c, pl.ds(r0, rpt)])

  return edge_pass


def _tc1_body(x_ref, w1_ref, b1_ref, wu_ref, x1_ref, u_ref):
  xb = x_ref[...]
  x1_ref[...] = jnp.maximum(xb @ w1_ref[...] + b1_ref[...], 0.0)
  u_ref[...] = xb @ wu_ref[...]


def _tc2_body(degp_ref, u_ref, t1_ref, dinv_ref):
  deg = degp_ref[0] + degp_ref[1] + 1.0
  dinv = 1.0 / jnp.sqrt(deg)
  dinv_ref[...] = dinv
  t1_ref[...] = u_ref[...] * dinv[:, 0:1]


def _tc3_body(z1_ref, t1_ref, dinv_ref, b2_ref, x2_ref, t2_ref):
  dinv = dinv_ref[:, 0:1]
  h = t1_ref.shape[1] // 2
  zsum = z1_ref[0] + z1_ref[1] + t1_ref[...]
  x2_ref[...] = jnp.maximum(zsum[:, :h] * dinv + b2_ref[...], 0.0)
  t2_ref[...] = zsum[:, h:] * (dinv * dinv)


def _tc4_body(z2_ref, t2_ref, dinv_ref, x1_ref, x2_ref, b3_ref,
              wl1_ref, wl2_ref, wl3_ref, bl_ref, out_ref):
  dinv = dinv_ref[:, 0:1]
  y3 = (z2_ref[0] + z2_ref[1] + t2_ref[...]) * dinv
  x3 = jnp.maximum(y3 + b3_ref[...], 0.0)
  z = (x1_ref[...] @ wl1_ref[...] + x2_ref[...] @ wl2_ref[...]
       + x3 @ wl3_ref[...] + bl_ref[...])
  m = jnp.max(z, axis=1, keepdims=True)
  out_ref[...] = z - (m + jnp.log(jnp.sum(jnp.exp(z - m), axis=1, keepdims=True)))


def _full(shape):
  return pl.BlockSpec(shape, lambda i: (0,) * len(shape))


def kernel(x, edge_index, W1, b1, W2, b2, W3, b3, Wl, bl):
  n, d_feat = x.shape
  hidden = W1.shape[1]
  n_classes = Wl.shape[1]
  e = edge_index.shape[1]

  n_pad = ((n + 2047) // 2048) * 2048
  egrp = NS * CHUNK * K
  e_pad = ((e + egrp - 1) // egrp) * egrp
  rpt = n_pad // NS
  total_chunks = e_pad // CHUNK

  src = jnp.concatenate([edge_index[0], jnp.full((e_pad - e + egrp,), n, jnp.int32)])
  dst = jnp.concatenate([edge_index[1], jnp.full((e_pad - e + egrp,), n, jnp.int32)])
  src = src.reshape(total_chunks + NS * K, CHUNK)
  dst = dst.reshape(total_chunks + NS * K, CHUNK)

  xp = jnp.concatenate([x, jnp.zeros((n_pad - n, d_feat), x.dtype)])
  wu = jnp.concatenate([W2, W3], axis=1)
  ones16 = jnp.ones((CHUNK, 16), jnp.float32)
  z16 = jnp.zeros((rpt, 16), jnp.float32)
  z32 = jnp.zeros((rpt, 32), jnp.float32)
  z64 = jnp.zeros((rpt, 64), jnp.float32)

  rblk = 1024
  ngrid = n_pad // rblk

  x1p, up = pl.pallas_call(
      _tc1_body,
      grid=(ngrid,),
      in_specs=[
          pl.BlockSpec((rblk, d_feat), lambda i: (i, 0)),
          _full((d_feat, hidden)),
          _full((1, hidden)),
          _full((d_feat, 2 * hidden)),
      ],
      out_specs=[
          pl.BlockSpec((rblk, hidden), lambda i: (i, 0)),
          pl.BlockSpec((rblk, 2 * hidden), lambda i: (i, 0)),
      ],
      out_shape=[
          jax.ShapeDtypeStruct((n_pad, hidden), jnp.float32),
          jax.ShapeDtypeStruct((n_pad, 2 * hidden), jnp.float32),
      ],
  )(xp, W1, b1.reshape(1, hidden), wu)

  degp = _make_deg_pass(n_pad, e_pad)(dst, ones16, z16)

  t1p, dinvb = pl.pallas_call(
      _tc2_body,
      grid=(ngrid,),
      in_specs=[
          pl.BlockSpec((2, rblk, 16), lambda i: (0, i, 0)),
          pl.BlockSpec((rblk, 2 * hidden), lambda i: (i, 0)),
      ],
      out_specs=[
          pl.BlockSpec((rblk, 2 * hidden), lambda i: (i, 0)),
          pl.BlockSpec((rblk, 16), lambda i: (i, 0)),
      ],
      out_shape=[
          jax.ShapeDtypeStruct((n_pad, 2 * hidden), jnp.float32),
          jax.ShapeDtypeStruct((n_pad, 16), jnp.float32),
      ],
  )(degp, up)

  z1p = _make_edge_pass(n_pad, e_pad, 2 * hidden)(t1p, src, dst, z64)

  x2p, t2p = pl.pallas_call(
      _tc3_body,
      grid=(ngrid,),
      in_specs=[
          pl.BlockSpec((2, rblk, 2 * hidden), lambda i: (0, i, 0)),
          pl.BlockSpec((rblk, 2 * hidden), lambda i: (i, 0)),
          pl.BlockSpec((rblk, 16), lambda i: (i, 0)),
          _full((1, hidden)),
      ],
      out_specs=[
          pl.BlockSpec((rblk, hidden), lambda i: (i, 0)),
          pl.BlockSpec((rblk, hidden), lambda i: (i, 0)),
      ],
      out_shape=[
          jax.ShapeDtypeStruct((n_pad, hidden), jnp.float32),
          jax.ShapeDtypeStruct((n_pad, hidden), jnp.float32),
      ],
  )(z1p, t1p, dinvb, b2.reshape(1, hidden))

  z2p = _make_edge_pass(n_pad, e_pad, hidden)(t2p, src, dst, z32)

  rblk4 = 1000
  ngrid4 = n // rblk4
  out = pl.pallas_call(
      _tc4_body,
      grid=(ngrid4,),
      in_specs=[
          pl.BlockSpec((2, rblk4, hidden), lambda i: (0, i, 0)),
          pl.BlockSpec((rblk4, hidden), lambda i: (i, 0)),
          pl.BlockSpec((rblk4, 16), lambda i: (i, 0)),
          pl.BlockSpec((rblk4, hidden), lambda i: (i, 0)),
          pl.BlockSpec((rblk4, hidden), lambda i: (i, 0)),
          _full((1, hidden)),
          _full((hidden, n_classes)),
          _full((hidden, n_classes)),
          _full((hidden, n_classes)),
          _full((1, n_classes)),
      ],
      out_specs=pl.BlockSpec((rblk4, n_classes), lambda i: (i, 0)),
      out_shape=jax.ShapeDtypeStruct((n, n_classes), jnp.float32),
  )(z2p, t2p, dinvb, x1p, x2p, b3.reshape(1, hidden),
    Wl[:hidden], Wl[hidden:2 * hidden], Wl[2 * hidden:], bl.reshape(1, n_classes))

  return out


# speedup vs baseline: 1.3376x; 1.0175x over previous
import functools

import jax
import jax.numpy as jnp
from jax import lax
from jax.experimental import pallas as pl
from jax.experimental.pallas import tpu as pltpu
from jax.experimental.pallas import tpu_sc as plsc

NC = 2
NS = 16
NW = NC * NS
CHUNK = 128

_MESH = plsc.VectorSubcoreMesh(
    core_axis_name="c", subcore_axis_name="s", num_cores=NC, num_subcores=NS
)


K = 8
CORE0_SHARE = 0.9


def _slab(c, s, n0, n1):
  is0 = c == 0
  nmine = jnp.where(is0, n0, n1)
  chunk0 = jnp.where(is0, s * n0, NS * n0 + s * n1)
  groups = jnp.where(is0, n0 // K, n1 // K)
  return chunk0, nmine, groups


def _splits(e_pad):
  per_pair = e_pad // (CHUNK * NS)
  n0 = max(K, (int(per_pair * CORE0_SHARE) // K) * K)
  n1 = per_pair - n0
  return n0, n1


SLAB = 64


def _make_deg_pass(n_pad, e_pad):
  n0, n1 = _splits(e_pad)
  rpt = n_pad // NS

  @functools.partial(
      pl.kernel,
      out_type=jax.ShapeDtypeStruct((NC, n_pad, 16), jnp.float32),
      mesh=_MESH,
      scratch_types=[
          pltpu.VMEM((SLAB, CHUNK), jnp.int32),
          pltpu.VMEM((CHUNK, 16), jnp.float32),
          pltpu.VMEM_SHARED((n_pad, 16), jnp.float32),
          pltpu.SemaphoreType.DMA,
      ],
      compiler_params=pltpu.CompilerParams(use_tc_tiling_on_sc=False),
  )
  def deg_pass(dst_hbm, ones_hbm, zeros_hbm, out_hbm, didx_a, ones_v, acc_sh, sem):
    c = lax.axis_index("c")
    s = lax.axis_index("s")
    chunk0, nmine, _ = _slab(c, s, n0, n1)
    r0 = s * rpt
    pltpu.sync_copy(zeros_hbm, acc_sh.at[pl.ds(r0, rpt)])
    pltpu.sync_copy(ones_hbm, ones_v)
    plsc.subcore_barrier()

    @pl.loop(0, (nmine + SLAB - 1) // SLAB)
    def _half(h):
      sl0 = chunk0 + h * SLAB
      pltpu.sync_copy(dst_hbm.at[pl.ds(sl0, SLAB)], didx_a)

      @pl.loop(0, jnp.minimum(SLAB, nmine - h * SLAB) // K)
      def _grp(g):
        base = g * K
        cps = [
            pltpu.async_copy(ones_v, acc_sh.at[didx_a.at[base + b]], sem,
                             add=True)
            for b in range(K)
        ]
        for cp in cps:
          cp.wait()

    plsc.subcore_barrier()
    pltpu.sync_copy(acc_sh.at[pl.ds(r0, rpt)], out_hbm.at[c, pl.ds(r0, rpt)])

  return deg_pass


def _make_edge_pass(n_pad, e_pad, width):
  n0, n1 = _splits(e_pad)
  rpt = n_pad // NS

  @functools.partial(
      pl.kernel,
      out_type=jax.ShapeDtypeStruct((NC, n_pad, width), jnp.float32),
      mesh=_MESH,
      scratch_types=[
          pltpu.VMEM((SLAB, CHUNK), jnp.int32),
          pltpu.VMEM((SLAB, CHUNK), jnp.int32),
          pltpu.VMEM((K, CHUNK, width), jnp.float32),
          pltpu.VMEM_SHARED((n_pad, width), jnp.float32),
          pltpu.SemaphoreType.DMA,
          pltpu.SemaphoreType.DMA,
      ],
      compiler_params=pltpu.CompilerParams(use_tc_tiling_on_sc=False),
  )
  def edge_pass(t_hbm, src_hbm, dst_hbm, zeros_hbm, out_hbm,
                sidx_a, didx_a, rows_v, acc_sh, sem_g, sem_s):
    c = lax.axis_index("c")
    s = lax.axis_index("s")
    chunk0, nmine, _ = _slab(c, s, n0, n1)
    r0 = s * rpt
    pltpu.sync_copy(zeros_hbm, acc_sh.at[pl.ds(r0, rpt)])
    plsc.subcore_barrier()

    @pl.loop(0, (nmine + SLAB - 1) // SLAB)
    def _half(h):
      sl0 = chunk0 + h * SLAB
      pltpu.sync_copy(src_hbm.at[pl.ds(sl0, SLAB)], sidx_a)
      pltpu.sync_copy(dst_hbm.at[pl.ds(sl0, SLAB)], didx_a)

      @pl.loop(0, jnp.minimum(SLAB, nmine - h * SLAB) // K)
      def _grp(g):
        base = g * K
        gcs = [
            pltpu.async_copy(t_hbm.at[sidx_a.at[base + b]], rows_v.at[b], sem_g)
            for b in range(K)
        ]
        for cp in gcs:
          cp.wait()
        scs = [
            pltpu.async_copy(rows_v.at[b], acc_sh.at[didx_a.at[base + b]],
                             sem_s, add=True)
            for b in range(K)
        ]
        for cp in scs:
          cp.wait()

    plsc.subcore_barrier()
    pltpu.sync_copy(acc_sh.at[pl.ds(r0, rpt)], out_hbm.at[c, pl.ds(r0, rpt)])

  return edge_pass


def _tc1_body(x_ref, w1_ref, b1_ref, wu_ref, x1_ref, u_ref):
  xb = x_ref[...]
  x1_ref[...] = jnp.maximum(xb @ w1_ref[...] + b1_ref[...], 0.0)
  u_ref[...] = xb @ wu_ref[...]


def _tc2_body(degp_ref, u_ref, t1_ref, dinv_ref):
  deg = degp_ref[0] + degp_ref[1] + 1.0
  dinv = 1.0 / jnp.sqrt(deg)
  dinv_ref[...] = dinv
  t1_ref[...] = u_ref[...] * dinv[:, 0:1]


def _tc3_body(z1_ref, t1_ref, dinv_ref, b2_ref, x2_ref, t2_ref):
  dinv = dinv_ref[:, 0:1]
  h = t1_ref.shape[1] // 2
  zsum = z1_ref[0] + z1_ref[1] + t1_ref[...]
  x2_ref[...] = jnp.maximum(zsum[:, :h] * dinv + b2_ref[...], 0.0)
  t2_ref[...] = zsum[:, h:] * (dinv * dinv)


def _tc4_body(z2_ref, t2_ref, dinv_ref, x1_ref, x2_ref, b3_ref,
              wl1_ref, wl2_ref, wl3_ref, bl_ref, out_ref):
  dinv = dinv_ref[:, 0:1]
  y3 = (z2_ref[0] + z2_ref[1] + t2_ref[...]) * dinv
  x3 = jnp.maximum(y3 + b3_ref[...], 0.0)
  z = (x1_ref[...] @ wl1_ref[...] + x2_ref[...] @ wl2_ref[...]
       + x3 @ wl3_ref[...] + bl_ref[...])
  m = jnp.max(z, axis=1, keepdims=True)
  out_ref[...] = z - (m + jnp.log(jnp.sum(jnp.exp(z - m), axis=1, keepdims=True)))


def _full(shape):
  return pl.BlockSpec(shape, lambda i: (0,) * len(shape))


def kernel(x, edge_index, W1, b1, W2, b2, W3, b3, Wl, bl):
  n, d_feat = x.shape
  hidden = W1.shape[1]
  n_classes = Wl.shape[1]
  e = edge_index.shape[1]

  n_pad = ((n + 2047) // 2048) * 2048
  egrp = NS * CHUNK * K
  e_pad = ((e + egrp - 1) // egrp) * egrp
  rpt = n_pad // NS
  total_chunks = e_pad // CHUNK

  src = jnp.concatenate([edge_index[0], jnp.full((e_pad - e + egrp,), n, jnp.int32)])
  dst = jnp.concatenate([edge_index[1], jnp.full((e_pad - e + egrp,), n, jnp.int32)])
  src = src.reshape(total_chunks + NS * K, CHUNK)
  dst = dst.reshape(total_chunks + NS * K, CHUNK)

  xp = jnp.concatenate([x, jnp.zeros((n_pad - n, d_feat), x.dtype)])
  wu = jnp.concatenate([W2, W3], axis=1)
  ones16 = jnp.ones((CHUNK, 16), jnp.float32)
  z16 = jnp.zeros((rpt, 16), jnp.float32)
  z32 = jnp.zeros((rpt, 32), jnp.float32)
  z64 = jnp.zeros((rpt, 64), jnp.float32)

  rblk = 1024
  ngrid = n_pad // rblk

  x1p, up = pl.pallas_call(
      _tc1_body,
      grid=(ngrid,),
      in_specs=[
          pl.BlockSpec((rblk, d_feat), lambda i: (i, 0)),
          _full((d_feat, hidden)),
          _full((1, hidden)),
          _full((d_feat, 2 * hidden)),
      ],
      out_specs=[
          pl.BlockSpec((rblk, hidden), lambda i: (i, 0)),
          pl.BlockSpec((rblk, 2 * hidden), lambda i: (i, 0)),
      ],
      out_shape=[
          jax.ShapeDtypeStruct((n_pad, hidden), jnp.float32),
          jax.ShapeDtypeStruct((n_pad, 2 * hidden), jnp.float32),
      ],
  )(xp, W1, b1.reshape(1, hidden), wu)

  degp = _make_deg_pass(n_pad, e_pad)(dst, ones16, z16)

  t1p, dinvb = pl.pallas_call(
      _tc2_body,
      grid=(ngrid,),
      in_specs=[
          pl.BlockSpec((2, rblk, 16), lambda i: (0, i, 0)),
          pl.BlockSpec((rblk, 2 * hidden), lambda i: (i, 0)),
      ],
      out_specs=[
          pl.BlockSpec((rblk, 2 * hidden), lambda i: (i, 0)),
          pl.BlockSpec((rblk, 16), lambda i: (i, 0)),
      ],
      out_shape=[
          jax.ShapeDtypeStruct((n_pad, 2 * hidden), jnp.float32),
          jax.ShapeDtypeStruct((n_pad, 16), jnp.float32),
      ],
  )(degp, up)

  z1p = _make_edge_pass(n_pad, e_pad, 2 * hidden)(t1p, src, dst, z64)

  x2p, t2p = pl.pallas_call(
      _tc3_body,
      grid=(ngrid,),
      in_specs=[
          pl.BlockSpec((2, rblk, 2 * hidden), lambda i: (0, i, 0)),
          pl.BlockSpec((rblk, 2 * hidden), lambda i: (i, 0)),
          pl.BlockSpec((rblk, 16), lambda i: (i, 0)),
          _full((1, hidden)),
      ],
      out_specs=[
          pl.BlockSpec((rblk, hidden), lambda i: (i, 0)),
          pl.BlockSpec((rblk, hidden), lambda i: (i, 0)),
      ],
      out_shape=[
          jax.ShapeDtypeStruct((n_pad, hidden), jnp.float32),
          jax.ShapeDtypeStruct((n_pad, hidden), jnp.float32),
      ],
  )(z1p, t1p, dinvb, b2.reshape(1, hidden))

  z2p = _make_edge_pass(n_pad, e_pad, hidden)(t2p, src, dst, z32)

  rblk4 = 1000
  ngrid4 = n // rblk4
  out = pl.pallas_call(
      _tc4_body,
      grid=(ngrid4,),
      in_specs=[
          pl.BlockSpec((2, rblk4, hidden), lambda i: (0, i, 0)),
          pl.BlockSpec((rblk4, hidden), lambda i: (i, 0)),
          pl.BlockSpec((rblk4, 16), lambda i: (i, 0)),
          pl.BlockSpec((rblk4, hidden), lambda i: (i, 0)),
          pl.BlockSpec((rblk4, hidden), lambda i: (i, 0)),
          _full((1, hidden)),
          _full((hidden, n_classes)),
          _full((hidden, n_classes)),
          _full((hidden, n_classes)),
          _full((1, n_classes)),
      ],
      out_specs=pl.BlockSpec((rblk4, n_classes), lambda i: (i, 0)),
      out_shape=jax.ShapeDtypeStruct((n, n_classes), jnp.float32),
  )(z2p, t2p, dinvb, x1p, x2p, b3.reshape(1, hidden),
    Wl[:hidden], Wl[hidden:2 * hidden], Wl[2 * hidden:], bl.reshape(1, n_classes))

  return out


# speedup vs baseline: 1.3647x; 1.0203x over previous
import functools

import jax
import jax.numpy as jnp
from jax import lax
from jax.experimental import pallas as pl
from jax.experimental.pallas import tpu as pltpu
from jax.experimental.pallas import tpu_sc as plsc

NC = 2
NS = 16
NW = NC * NS
CHUNK = 128

_MESH = plsc.VectorSubcoreMesh(
    core_axis_name="c", subcore_axis_name="s", num_cores=NC, num_subcores=NS
)


K = 8
CORE0_SHARE = 0.9


def _slab(c, s, n0, n1):
  is0 = c == 0
  nmine = jnp.where(is0, n0, n1)
  chunk0 = jnp.where(is0, s * n0, NS * n0 + s * n1)
  groups = jnp.where(is0, n0 // K, n1 // K)
  return chunk0, nmine, groups


def _splits(e_pad):
  per_pair = e_pad // (CHUNK * NS)
  n0 = max(K, (int(per_pair * CORE0_SHARE) // K) * K)
  n1 = per_pair - n0
  return n0, n1


SLAB = 64


def _make_deg_pass(n_pad, e_pad):
  n0, n1 = _splits(e_pad)
  rpt = n_pad // NS

  @functools.partial(
      pl.kernel,
      out_type=jax.ShapeDtypeStruct((NC, n_pad, 16), jnp.float32),
      mesh=_MESH,
      scratch_types=[
          pltpu.VMEM((SLAB, CHUNK), jnp.int32),
          pltpu.VMEM((CHUNK, 16), jnp.float32),
          pltpu.VMEM((CHUNK, 16), jnp.float32),
          pltpu.VMEM_SHARED((n_pad, 16), jnp.float32),
          pltpu.SemaphoreType.DMA,
      ],
      compiler_params=pltpu.CompilerParams(use_tc_tiling_on_sc=False),
  )
  def deg_pass(dst_hbm, ones_hbm, out_hbm, didx_a, ones_v, zrow_v, acc_sh, sem):
    c = lax.axis_index("c")
    s = lax.axis_index("s")
    chunk0, nmine, _ = _slab(c, s, n0, n1)
    r0 = s * rpt
    zv = jnp.zeros((16,), jnp.float32)

    @pl.loop(0, CHUNK)
    def _zf(i):
      zrow_v[i] = zv

    zcs = [
        pltpu.async_copy(zrow_v, acc_sh.at[pl.ds(r0 + r * CHUNK, CHUNK)], sem)
        for r in range(rpt // CHUNK)
    ]
    pltpu.sync_copy(ones_hbm, ones_v)
    for cp in zcs:
      cp.wait()
    plsc.subcore_barrier()

    @pl.loop(0, (nmine + SLAB - 1) // SLAB)
    def _half(h):
      sl0 = chunk0 + h * SLAB
      pltpu.sync_copy(dst_hbm.at[pl.ds(sl0, SLAB)], didx_a)

      @pl.loop(0, jnp.minimum(SLAB, nmine - h * SLAB) // K)
      def _grp(g):
        base = g * K
        cps = [
            pltpu.async_copy(ones_v, acc_sh.at[didx_a.at[base + b]], sem,
                             add=True)
            for b in range(K)
        ]
        for cp in cps:
          cp.wait()

    plsc.subcore_barrier()
    pltpu.sync_copy(acc_sh.at[pl.ds(r0, rpt)], out_hbm.at[c, pl.ds(r0, rpt)])

  return deg_pass


def _make_edge_pass(n_pad, e_pad, width):
  n0, n1 = _splits(e_pad)
  rpt = n_pad // NS

  @functools.partial(
      pl.kernel,
      out_type=jax.ShapeDtypeStruct((NC, n_pad, width), jnp.float32),
      mesh=_MESH,
      scratch_types=[
          pltpu.VMEM((SLAB, CHUNK), jnp.int32),
          pltpu.VMEM((SLAB, CHUNK), jnp.int32),
          pltpu.VMEM((K, CHUNK, width), jnp.float32),
          pltpu.VMEM_SHARED((n_pad, width), jnp.float32),
          pltpu.SemaphoreType.DMA,
          pltpu.SemaphoreType.DMA,
      ],
      compiler_params=pltpu.CompilerParams(use_tc_tiling_on_sc=False),
  )
  def edge_pass(t_hbm, src_hbm, dst_hbm, out_hbm,
                sidx_a, didx_a, rows_v, acc_sh, sem_g, sem_s):
    c = lax.axis_index("c")
    s = lax.axis_index("s")
    chunk0, nmine, _ = _slab(c, s, n0, n1)
    r0 = s * rpt
    zv = jnp.zeros((16,), jnp.float32)

    @pl.loop(0, CHUNK)
    def _zf(i):
      for q in range(width // 16):
        rows_v[0, i, pl.ds(q * 16, 16)] = zv

    zcs = [
        pltpu.async_copy(rows_v.at[0], acc_sh.at[pl.ds(r0 + r * CHUNK, CHUNK)],
                         sem_s)
        for r in range(rpt // CHUNK)
    ]
    for cp in zcs:
      cp.wait()
    plsc.subcore_barrier()

    @pl.loop(0, (nmine + SLAB - 1) // SLAB)
    def _half(h):
      sl0 = chunk0 + h * SLAB
      pltpu.sync_copy(src_hbm.at[pl.ds(sl0, SLAB)], sidx_a)
      pltpu.sync_copy(dst_hbm.at[pl.ds(sl0, SLAB)], didx_a)

      @pl.loop(0, jnp.minimum(SLAB, nmine - h * SLAB) // K)
      def _grp(g):
        base = g * K
        gcs = [
            pltpu.async_copy(t_hbm.at[sidx_a.at[base + b]], rows_v.at[b], sem_g)
            for b in range(K)
        ]
        for cp in gcs:
          cp.wait()
        scs = [
            pltpu.async_copy(rows_v.at[b], acc_sh.at[didx_a.at[base + b]],
                             sem_s, add=True)
            for b in range(K)
        ]
        for cp in scs:
          cp.wait()

    plsc.subcore_barrier()
    pltpu.sync_copy(acc_sh.at[pl.ds(r0, rpt)], out_hbm.at[c, pl.ds(r0, rpt)])

  return edge_pass


def _tc1_body(x_ref, w1_ref, b1_ref, wu_ref, x1_ref, u_ref):
  xb = x_ref[...]
  x1_ref[...] = jnp.maximum(xb @ w1_ref[...] + b1_ref[...], 0.0)
  u_ref[...] = xb @ wu_ref[...]


def _tc2_body(degp_ref, u_ref, t1_ref, dinv_ref):
  deg = degp_ref[0] + degp_ref[1] + 1.0
  dinv = 1.0 / jnp.sqrt(deg)
  dinv_ref[...] = dinv
  t1_ref[...] = u_ref[...] * dinv[:, 0:1]


def _tc3_body(z1_ref, t1_ref, dinv_ref, b2_ref, x2_ref, t2_ref):
  dinv = dinv_ref[:, 0:1]
  h = t1_ref.shape[1] // 2
  zsum = z1_ref[0] + z1_ref[1] + t1_ref[...]
  x2_ref[...] = jnp.maximum(zsum[:, :h] * dinv + b2_ref[...], 0.0)
  t2_ref[...] = zsum[:, h:] * (dinv * dinv)


def _tc4_body(z2_ref, t2_ref, dinv_ref, x1_ref, x2_ref, b3_ref,
              wl1_ref, wl2_ref, wl3_ref, bl_ref, out_ref):
  dinv = dinv_ref[:, 0:1]
  y3 = (z2_ref[0] + z2_ref[1] + t2_ref[...]) * dinv
  x3 = jnp.maximum(y3 + b3_ref[...], 0.0)
  z = (x1_ref[...] @ wl1_ref[...] + x2_ref[...] @ wl2_ref[...]
       + x3 @ wl3_ref[...] + bl_ref[...])
  m = jnp.max(z, axis=1, keepdims=True)
  out_ref[...] = z - (m + jnp.log(jnp.sum(jnp.exp(z - m), axis=1, keepdims=True)))


def _full(shape):
  return pl.BlockSpec(shape, lambda i: (0,) * len(shape))


def kernel(x, edge_index, W1, b1, W2, b2, W3, b3, Wl, bl):
  n, d_feat = x.shape
  hidden = W1.shape[1]
  n_classes = Wl.shape[1]
  e = edge_index.shape[1]

  n_pad = ((n + 2047) // 2048) * 2048
  egrp = NS * CHUNK * K
  e_pad = ((e + egrp - 1) // egrp) * egrp
  rpt = n_pad // NS
  total_chunks = e_pad // CHUNK

  src = jnp.concatenate([edge_index[0], jnp.full((e_pad - e + egrp,), n, jnp.int32)])
  dst = jnp.concatenate([edge_index[1], jnp.full((e_pad - e + egrp,), n, jnp.int32)])
  src = src.reshape(total_chunks + NS * K, CHUNK)
  dst = dst.reshape(total_chunks + NS * K, CHUNK)

  xp = jnp.concatenate([x, jnp.zeros((n_pad - n, d_feat), x.dtype)])
  wu = jnp.concatenate([W2, W3], axis=1)
  ones16 = jnp.ones((CHUNK, 16), jnp.float32)

  rblk = 1024
  ngrid = n_pad // rblk

  x1p, up = pl.pallas_call(
      _tc1_body,
      grid=(ngrid,),
      in_specs=[
          pl.BlockSpec((rblk, d_feat), lambda i: (i, 0)),
          _full((d_feat, hidden)),
          _full((1, hidden)),
          _full((d_feat, 2 * hidden)),
      ],
      out_specs=[
          pl.BlockSpec((rblk, hidden), lambda i: (i, 0)),
          pl.BlockSpec((rblk, 2 * hidden), lambda i: (i, 0)),
      ],
      out_shape=[
          jax.ShapeDtypeStruct((n_pad, hidden), jnp.float32),
          jax.ShapeDtypeStruct((n_pad, 2 * hidden), jnp.float32),
      ],
  )(xp, W1, b1.reshape(1, hidden), wu)

  degp = _make_deg_pass(n_pad, e_pad)(dst, ones16)

  t1p, dinvb = pl.pallas_call(
      _tc2_body,
      grid=(ngrid,),
      in_specs=[
          pl.BlockSpec((2, rblk, 16), lambda i: (0, i, 0)),
          pl.BlockSpec((rblk, 2 * hidden), lambda i: (i, 0)),
      ],
      out_specs=[
          pl.BlockSpec((rblk, 2 * hidden), lambda i: (i, 0)),
          pl.BlockSpec((rblk, 16), lambda i: (i, 0)),
      ],
      out_shape=[
          jax.ShapeDtypeStruct((n_pad, 2 * hidden), jnp.float32),
          jax.ShapeDtypeStruct((n_pad, 16), jnp.float32),
      ],
  )(degp, up)

  z1p = _make_edge_pass(n_pad, e_pad, 2 * hidden)(t1p, src, dst)

  x2p, t2p = pl.pallas_call(
      _tc3_body,
      grid=(ngrid,),
      in_specs=[
          pl.BlockSpec((2, rblk, 2 * hidden), lambda i: (0, i, 0)),
          pl.BlockSpec((rblk, 2 * hidden), lambda i: (i, 0)),
          pl.BlockSpec((rblk, 16), lambda i: (i, 0)),
          _full((1, hidden)),
      ],
      out_specs=[
          pl.BlockSpec((rblk, hidden), lambda i: (i, 0)),
          pl.BlockSpec((rblk, hidden), lambda i: (i, 0)),
      ],
      out_shape=[
          jax.ShapeDtypeStruct((n_pad, hidden), jnp.float32),
          jax.ShapeDtypeStruct((n_pad, hidden), jnp.float32),
      ],
  )(z1p, t1p, dinvb, b2.reshape(1, hidden))

  z2p = _make_edge_pass(n_pad, e_pad, hidden)(t2p, src, dst)

  rblk4 = 1000
  ngrid4 = n // rblk4
  out = pl.pallas_call(
      _tc4_body,
      grid=(ngrid4,),
      in_specs=[
          pl.BlockSpec((2, rblk4, hidden), lambda i: (0, i, 0)),
          pl.BlockSpec((rblk4, hidden), lambda i: (i, 0)),
          pl.BlockSpec((rblk4, 16), lambda i: (i, 0)),
          pl.BlockSpec((rblk4, hidden), lambda i: (i, 0)),
          pl.BlockSpec((rblk4, hidden), lambda i: (i, 0)),
          _full((1, hidden)),
          _full((hidden, n_classes)),
          _full((hidden, n_classes)),
          _full((hidden, n_classes)),
          _full((1, n_classes)),
      ],
      out_specs=pl.BlockSpec((rblk4, n_classes), lambda i: (i, 0)),
      out_shape=jax.ShapeDtypeStruct((n, n_classes), jnp.float32),
  )(z2p, t2p, dinvb, x1p, x2p, b3.reshape(1, hidden),
    Wl[:hidden], Wl[hidden:2 * hidden], Wl[2 * hidden:], bl.reshape(1, n_classes))

  return out
